# baseline (speedup 1.0000x reference)
import functools

import jax
import jax.numpy as jnp
from jax import lax
from jax.experimental import pallas as pl
from jax.experimental.pallas import tpu as pltpu
from jax.experimental.pallas import tpu_sc as plsc

DENSE_N = 13
SPARSE_F = 26
VOCAB = 1000
EMB = 8
HID = 256
SEQ_N = 16
SEQ_LEN = 30
EXPERTS = 8
CHUNK = HID // EXPERTS

BB = 2048
PACK = 16
NC, NS = 2, 16
NW = NC * NS


def _sc_gather_call(emb2, emb1, idx_flat, n_idx):
    rows_per_w = n_idx // NW
    mesh = plsc.VectorSubcoreMesh(core_axis_name="c", subcore_axis_name="s")

    @functools.partial(
        pl.kernel, mesh=mesh,
        out_type=(jax.ShapeDtypeStruct((n_idx, EMB), jnp.float32),
                  jax.ShapeDtypeStruct((n_idx,), jnp.float32)),
        compiler_params=pltpu.CompilerParams(use_tc_tiling_on_sc=False,
                                             needs_layout_passes=False),
        scratch_types=[
            pltpu.VMEM((rows_per_w,), jnp.int32),
            pltpu.VMEM((rows_per_w, EMB), jnp.float32),
            pltpu.VMEM((rows_per_w,), jnp.float32),
            pltpu.VMEM((SPARSE_F * VOCAB,), jnp.float32),
            pltpu.SemaphoreType.DMA,
        ],
    )
    def _k(emb2_hbm, emb1_hbm, idx_hbm, out2_hbm, out1_hbm,
           idx_v, rows2_v, out1_v, emb1_v, sem2):
        wid = lax.axis_index("s") * NC + lax.axis_index("c")
        base = wid * rows_per_w
        pltpu.sync_copy(idx_hbm.at[pl.ds(base, rows_per_w)], idx_v)
        c2 = pltpu.async_copy(emb2_hbm.at[idx_v], rows2_v, sem2)
        pltpu.sync_copy(emb1_hbm, emb1_v)

        def _body(i, _):
            iv = idx_v[pl.ds(i * 16, 16)]
            out1_v[pl.ds(i * 16, 16)] = plsc.load_gather(emb1_v, [iv])
            return 0

        lax.fori_loop(0, rows_per_w // 16, _body, 0)
        c2.wait()
        pltpu.sync_copy(rows2_v, out2_hbm.at[pl.ds(base, rows_per_w)])
        pltpu.sync_copy(out1_v, out1_hbm.at[pl.ds(base, rows_per_w)])

    return _k(emb2, emb1, idx_flat)


def _tc_kernel(densex_ref, emb1v_ref, vflat_ref, xx27_ref, xx28_ref,
               xx29_ref, x29_ref,
               lin_w_ref, lin_b_ref, sel_ref,
               dnn_w0_ref, dnn_b0_ref, dnn_w1_ref, dnn_b1_ref,
               dnn_w2_ref, dnn_b2_ref,
               ww0_ref, b0_ref, w1cat_ref, b1_ref, wres_ref, bres_ref,
               fuse_w_ref, fb_ref, gaw_ref, gab_ref,
               gbxw_ref, gbxb_ref, gbsw_ref, gbsb_ref,
               outsel_ref, outb_ref,
               out_ref, smax_ref, smoid_ref):
    f32 = jnp.float32
    dot = functools.partial(jnp.dot, preferred_element_type=f32)
    dot_hi = functools.partial(jnp.dot, preferred_element_type=f32,
                               precision=jax.lax.Precision.HIGHEST)

    x = vflat_ref[...]
    sv = dot_hi(x, sel_ref[...])
    svq = dot_hi(x * x, sel_ref[...])
    second = 0.5 * (sv * sv - svq).sum(axis=-1, keepdims=True)
    first = (dot(densex_ref[...], lin_w_ref[...]) + lin_b_ref[...]
             + emb1v_ref[...].sum(axis=-1, keepdims=True))
    h = jnp.maximum(dot(x, dnn_w0_ref[...]) + dnn_b0_ref[...], 0.0)
    h = jnp.maximum(dot(h, dnn_w1_ref[...]) + dnn_b1_ref[...], 0.0)
    dnn = dot(h, dnn_w2_ref[...]) + dnn_b2_ref[...]
    p_e = (dnn + first) + second

    h27 = jnp.maximum(dot(xx27_ref[...], ww0_ref[...]) + b0_ref[...], 0.0)
    h28 = jnp.maximum(dot(xx28_ref[...], ww0_ref[...]) + b0_ref[...], 0.0)
    h29 = jnp.maximum(dot(xx29_ref[...], ww0_ref[...]) + b0_ref[...], 0.0)
    hcat = jnp.concatenate([h27, h28, h29], axis=1)
    h1 = jnp.maximum(dot(hcat, w1cat_ref[...]) + b1_ref[...], 0.0)
    res = dot(x29_ref[...], wres_ref[...]) + bres_ref[...]
    s_e = jnp.maximum(h1 + res, 0.0)

    cat = jnp.concatenate([s_e, p_e], axis=1)
    e = jnp.maximum(dot(cat, fuse_w_ref[...]) + fb_ref[...], 0.0)
    g_emb = jnp.tanh(dot(e, gaw_ref[...]) + gab_ref[...])
    g = dot(g_emb, gbxw_ref[...]) + gbxb_ref[...]
    gs = dot(g_emb, gbsw_ref[...]) + gbsb_ref[...]

    m = g.max(axis=-1, keepdims=True)
    ex = jnp.exp(g - m)
    smax_ref[...] = ex / ex.sum(axis=-1, keepdims=True)
    smoid_ref[...] = 1.0 / (1.0 + jnp.exp(-gs))

    iota = jax.lax.broadcasted_iota(jnp.int32, g.shape, 1)
    cand = jnp.where(g == m, iota, EXPERTS)
    fi = cand.min(axis=-1, keepdims=True)
    onehot = (iota == fi).astype(f32)
    chsum = dot(e, outsel_ref[...])
    out_ref[...] = ((chsum * onehot).sum(axis=-1, keepdims=True)
                    + outb_ref[...])


def kernel(data_p, data_stamp, data_s, params):
    del data_stamp
    p = params
    B = data_p.shape[0]
    f32 = jnp.float32

    dense_x = data_p[:, :DENSE_N]
    idx = (data_p[:, DENSE_N:].astype(jnp.int32)
           + (jnp.arange(SPARSE_F, dtype=jnp.int32) * VOCAB)[None, :])

    n_idx = B * SPARSE_F
    rows2, rows1 = _sc_gather_call(p['emb2'], p['emb1'][:, 0],
                                   idx.reshape(n_idx), n_idx)
    vflat = rows2.reshape(B, SPARSE_F * EMB)
    emb1v = rows1.reshape(B, SPARSE_F)

    w0 = p['tcn_w0']
    ww0 = jnp.concatenate([w0[:, :, k].T for k in range(3)], axis=0)
    w1 = p['tcn_w1']
    w1cat = jnp.concatenate([w1[:, :, k].T for k in range(3)], axis=0)
    wres = p['tcn_wres'][:, :, 0].T

    xt = lambda t: data_s[:, :, t]
    xx27 = jnp.concatenate([xt(25), xt(26), xt(27)], axis=1)
    xx28 = jnp.concatenate([xt(26), xt(27), xt(28)], axis=1)
    xx29 = jnp.concatenate([xt(27), xt(28), xt(29)], axis=1)
    x29 = xt(SEQ_LEN - 1)

    sel = jnp.tile(jnp.eye(EMB, dtype=f32), (SPARSE_F, 1))
    outsel = jnp.repeat(jnp.eye(EXPERTS, dtype=f32), CHUNK, axis=0) * p['out_w']

    row = lambda v: v.reshape(1, -1)
    weights = [
        p['lin_w'], row(p['lin_b']), sel,
        p['dnn_w0'], row(p['dnn_b0']), p['dnn_w1'], row(p['dnn_b1']),
        p['dnn_w2'], row(p['dnn_b2']),
        ww0, row(p['tcn_b0']), w1cat, row(p['tcn_b1']), wres, row(p['tcn_bres']),
        p['fuse_w'], row(p['fuse_b']),
        p['ga_w'], row(p['ga_b']),
        p['gb_smax_w'], row(p['gb_smax_b']),
        p['gb_smoid_w'], row(p['gb_smoid_b']),
        outsel, row(p['out_b']),
    ]

    grid = (B // BB,)
    data_spec = lambda d: pl.BlockSpec((BB, d), lambda i: (i, 0))
    w_spec = lambda w: pl.BlockSpec(w.shape, lambda i: (0, 0))

    out, smax, smoid = pl.pallas_call(
        _tc_kernel,
        grid=grid,
        in_specs=([data_spec(DENSE_N), data_spec(SPARSE_F),
                   data_spec(SPARSE_F * EMB), data_spec(3 * SEQ_N),
                   data_spec(3 * SEQ_N), data_spec(3 * SEQ_N),
                   data_spec(SEQ_N)]
                  + [w_spec(w) for w in weights]),
        out_specs=[data_spec(1), data_spec(EXPERTS), data_spec(EXPERTS)],
        out_shape=[jax.ShapeDtypeStruct((B, 1), f32),
                   jax.ShapeDtypeStruct((B, EXPERTS), f32),
                   jax.ShapeDtypeStruct((B, EXPERTS), f32)],
    )(dense_x, emb1v, vflat, xx27, xx28, xx29, x29, *weights)
    return out, smax, smoid


# speedup vs baseline: 1.1804x; 1.1804x over previous
import functools

import jax
import jax.numpy as jnp
from jax import lax
from jax.experimental import pallas as pl
from jax.experimental.pallas import tpu as pltpu
from jax.experimental.pallas import tpu_sc as plsc

DENSE_N = 13
SPARSE_F = 26
VOCAB = 1000
EMB = 8
HID = 256
SEQ_N = 16
SEQ_LEN = 30
EXPERTS = 8
CHUNK = HID // EXPERTS

BB = 1024
PACK = 16
NC, NS = 2, 16
NW = NC * NS


def _sc_gather_call(emb2, emb1, idx_flat, n_idx):
    rows_per_w = n_idx // NW
    mesh = plsc.VectorSubcoreMesh(core_axis_name="c", subcore_axis_name="s")

    @functools.partial(
        pl.kernel, mesh=mesh,
        out_type=(jax.ShapeDtypeStruct((n_idx, EMB), jnp.float32),
                  jax.ShapeDtypeStruct((n_idx,), jnp.float32)),
        compiler_params=pltpu.CompilerParams(use_tc_tiling_on_sc=False,
                                             needs_layout_passes=False),
        scratch_types=[
            pltpu.VMEM((rows_per_w,), jnp.int32),
            pltpu.VMEM((rows_per_w, EMB), jnp.float32),
            pltpu.VMEM((rows_per_w,), jnp.float32),
            pltpu.VMEM((SPARSE_F * VOCAB,), jnp.float32),
            pltpu.SemaphoreType.DMA,
        ],
    )
    def _k(emb2_hbm, emb1_hbm, idx_hbm, out2_hbm, out1_hbm,
           idx_v, rows2_v, out1_v, emb1_v, sem2):
        wid = lax.axis_index("s") * NC + lax.axis_index("c")
        base = wid * rows_per_w
        pltpu.sync_copy(idx_hbm.at[pl.ds(base, rows_per_w)], idx_v)
        c2 = pltpu.async_copy(emb2_hbm.at[idx_v], rows2_v, sem2)
        pltpu.sync_copy(emb1_hbm, emb1_v)

        def _body(i, _):
            iv = idx_v[pl.ds(i * 16, 16)]
            out1_v[pl.ds(i * 16, 16)] = plsc.load_gather(emb1_v, [iv])
            return 0

        lax.fori_loop(0, rows_per_w // 16, _body, 0)
        c2.wait()
        pltpu.sync_copy(rows2_v, out2_hbm.at[pl.ds(base, rows_per_w)])
        pltpu.sync_copy(out1_v, out1_hbm.at[pl.ds(base, rows_per_w)])

    return _k(emb2, emb1, idx_flat)


def _tc_kernel(densex_ref, emb1v_ref, vflat_ref, xw_ref,
               lin_w_ref, lin_b_ref, sel_ref,
               dnn_w0_ref, dnn_b0_ref, dnn_w1_ref, dnn_b1_ref,
               dnn_w2_ref, dnn_b2_ref,
               ww0_ref, b0_ref, w1cat_ref, b1_ref, wres_ref, bres_ref,
               fuse_w_ref, fb_ref, gaw_ref, gab_ref,
               gbxw_ref, gbxb_ref, gbsw_ref, gbsb_ref,
               outsel_ref, outb_ref,
               out_ref, smax_ref, smoid_ref):
    f32 = jnp.float32
    dot = functools.partial(jnp.dot, preferred_element_type=f32)
    dot_hi = functools.partial(jnp.dot, preferred_element_type=f32,
                               precision=jax.lax.Precision.HIGHEST)

    x = vflat_ref[...]
    sv = dot_hi(x, sel_ref[...])
    svq = dot_hi(x * x, sel_ref[...])
    second = 0.5 * (sv * sv - svq).sum(axis=-1, keepdims=True)
    first = (dot(densex_ref[...], lin_w_ref[...]) + lin_b_ref[...]
             + emb1v_ref[...].sum(axis=-1, keepdims=True))
    h = jnp.maximum(dot(x, dnn_w0_ref[...]) + dnn_b0_ref[...], 0.0)
    h = jnp.maximum(dot(h, dnn_w1_ref[...]) + dnn_b1_ref[...], 0.0)
    dnn = dot(h, dnn_w2_ref[...]) + dnn_b2_ref[...]
    p_e = (dnn + first) + second

    xw = xw_ref[...]
    h27 = jnp.maximum(dot(xw[:, 0:48], ww0_ref[...]) + b0_ref[...], 0.0)
    h28 = jnp.maximum(dot(xw[:, 16:64], ww0_ref[...]) + b0_ref[...], 0.0)
    h29 = jnp.maximum(dot(xw[:, 32:80], ww0_ref[...]) + b0_ref[...], 0.0)
    hcat = jnp.concatenate([h27, h28, h29], axis=1)
    h1 = jnp.maximum(dot(hcat, w1cat_ref[...]) + b1_ref[...], 0.0)
    res = dot(xw[:, 64:80], wres_ref[...]) + bres_ref[...]
    s_e = jnp.maximum(h1 + res, 0.0)

    cat = jnp.concatenate([s_e, p_e], axis=1)
    e = jnp.maximum(dot(cat, fuse_w_ref[...]) + fb_ref[...], 0.0)
    g_emb = jnp.tanh(dot(e, gaw_ref[...]) + gab_ref[...])
    g = dot(g_emb, gbxw_ref[...]) + gbxb_ref[...]
    gs = dot(g_emb, gbsw_ref[...]) + gbsb_ref[...]

    m = g.max(axis=-1, keepdims=True)
    ex = jnp.exp(g - m)
    smax_ref[...] = ex / ex.sum(axis=-1, keepdims=True)
    smoid_ref[...] = 1.0 / (1.0 + jnp.exp(-gs))

    iota = jax.lax.broadcasted_iota(jnp.int32, g.shape, 1)
    cand = jnp.where(g == m, iota, EXPERTS)
    fi = cand.min(axis=-1, keepdims=True)
    onehot = (iota == fi).astype(f32)
    chsum = dot(e, outsel_ref[...])
    out_ref[...] = ((chsum * onehot).sum(axis=-1, keepdims=True)
                    + outb_ref[...])


def kernel(data_p, data_stamp, data_s, params):
    del data_stamp
    p = params
    B = data_p.shape[0]
    f32 = jnp.float32

    dense_x = data_p[:, :DENSE_N]
    idx = (data_p[:, DENSE_N:].astype(jnp.int32)
           + (jnp.arange(SPARSE_F, dtype=jnp.int32) * VOCAB)[None, :])

    n_idx = B * SPARSE_F
    rows2, rows1 = _sc_gather_call(p['emb2'], p['emb1'][:, 0],
                                   idx.reshape(n_idx), n_idx)
    vflat = rows2.reshape(B, SPARSE_F * EMB)
    emb1v = rows1.reshape(B, SPARSE_F)

    w0 = p['tcn_w0']
    ww0 = jnp.concatenate([w0[:, :, k].T for k in range(3)], axis=0)
    w1 = p['tcn_w1']
    w1cat = jnp.concatenate([w1[:, :, k].T for k in range(3)], axis=0)
    wres = p['tcn_wres'][:, :, 0].T

    xw = (data_s[:, :, SEQ_LEN - 5:].transpose(0, 2, 1)
          .reshape(B, 5 * SEQ_N))

    sel = jnp.tile(jnp.eye(EMB, dtype=f32), (SPARSE_F, 1))
    outsel = jnp.repeat(jnp.eye(EXPERTS, dtype=f32), CHUNK, axis=0) * p['out_w']

    row = lambda v: v.reshape(1, -1)
    weights = [
        p['lin_w'], row(p['lin_b']), sel,
        p['dnn_w0'], row(p['dnn_b0']), p['dnn_w1'], row(p['dnn_b1']),
        p['dnn_w2'], row(p['dnn_b2']),
        ww0, row(p['tcn_b0']), w1cat, row(p['tcn_b1']), wres, row(p['tcn_bres']),
        p['fuse_w'], row(p['fuse_b']),
        p['ga_w'], row(p['ga_b']),
        p['gb_smax_w'], row(p['gb_smax_b']),
        p['gb_smoid_w'], row(p['gb_smoid_b']),
        outsel, row(p['out_b']),
    ]

    grid = (B // BB,)
    data_spec = lambda d: pl.BlockSpec((BB, d), lambda i: (i, 0))
    w_spec = lambda w: pl.BlockSpec(w.shape, lambda i: (0, 0))

    out, smax, smoid = pl.pallas_call(
        _tc_kernel,
        grid=grid,
        in_specs=([data_spec(DENSE_N), data_spec(SPARSE_F),
                   data_spec(SPARSE_F * EMB), data_spec(5 * SEQ_N)]
                  + [w_spec(w) for w in weights]),
        out_specs=[data_spec(1), data_spec(EXPERTS), data_spec(EXPERTS)],
        out_shape=[jax.ShapeDtypeStruct((B, 1), f32),
                   jax.ShapeDtypeStruct((B, EXPERTS), f32),
                   jax.ShapeDtypeStruct((B, EXPERTS), f32)],
    )(dense_x, emb1v, vflat, xw, *weights)
    return out, smax, smoid
